# baseline (speedup 1.0000x reference)
import functools
import math

import jax
import jax.numpy as jnp
from jax import lax
from jax.experimental import pallas as pl
from jax.experimental.pallas import tpu as pltpu
from jax.experimental.pallas import tpu_sc as plsc

VOCAB = 1000000
EMB_DIM = 64
PADDED = 128
BATCH = 4096
SEQ = 200

NC = 2
NS = 16
NW = NC * NS
LANES = 16

BBLK = BATCH // NW
NBUF = 3
SCALE = math.sqrt(EMB_DIM)

_mesh = plsc.VectorSubcoreMesh(
    core_axis_name="c", subcore_axis_name="s", num_cores=NC, num_subcores=NS
)


TCHK = 7813
TFULL = VOCAB // PADDED
TREM = VOCAB - TFULL * PADDED
FBUF = 3


@functools.partial(
    pl.kernel,
    out_type=jax.ShapeDtypeStruct((VOCAB, PADDED), jnp.float32),
    mesh=_mesh,
    compiler_params=pltpu.CompilerParams(
        use_tc_tiling_on_sc=True, needs_layout_passes=False
    ),
    scratch_types=[
        pltpu.VMEM((FBUF, EMB_DIM, PADDED), jnp.float32),
        pltpu.VMEM((FBUF, PADDED, PADDED), jnp.float32),
        pltpu.VMEM((EMB_DIM, TREM), jnp.float32),
        pltpu.VMEM((TREM, PADDED), jnp.float32),
        [pltpu.SemaphoreType.DMA] * FBUF,
        [pltpu.SemaphoreType.DMA] * FBUF,
    ],
)
def _format_table(tt_hbm, out_hbm, gbuf, tbuf, rbuf, rtbuf, gsems, ssems):
    wid = lax.axis_index("s") * NC + lax.axis_index("c")
    my_n = TFULL // NW + jnp.where(wid < TFULL % NW, 1, 0)

    iota = lax.iota(jnp.int32, LANES)

    def col0(k):
        return (wid + k * NW) * PADDED

    def transpose_block(bb, ngroups):
        def tblk(j0, carry2):
            rv = j0 * LANES + iota
            for d0 in range(EMB_DIM // LANES):
                for r in range(LANES):
                    cv = d0 * LANES + ((iota + r) & (LANES - 1))
                    v = plsc.load_gather(gbuf.at[bb], [cv, rv])
                    plsc.store_scatter(tbuf.at[bb], [rv, cv], v)
            return carry2

        lax.fori_loop(0, ngroups, tblk, 0, unroll=4)

    for b in range(FBUF):
        @pl.when(b < my_n)
        def _():
            pltpu.async_copy(
                tt_hbm.at[:, pl.ds(col0(b), PADDED)], gbuf.at[b], gsems[b]
            )

    def body(k, carry):
        for bb in range(FBUF):
            @pl.when(lax.rem(k, FBUF) == bb)
            def _():
                pltpu.make_async_copy(
                    tt_hbm.at[:, pl.ds(col0(k), PADDED)], gbuf.at[bb], gsems[bb]
                ).wait()

                @pl.when(k >= FBUF)
                def _():
                    pltpu.make_async_copy(
                        tbuf.at[bb],
                        out_hbm.at[pl.ds(col0(k - FBUF), PADDED)],
                        ssems[bb],
                    ).wait()

                transpose_block(bb, PADDED // LANES)

                pltpu.async_copy(
                    tbuf.at[bb], out_hbm.at[pl.ds(col0(k), PADDED)], ssems[bb]
                )

                @pl.when(k + FBUF < my_n)
                def _():
                    pltpu.async_copy(
                        tt_hbm.at[:, pl.ds(col0(k + FBUF), PADDED)],
                        gbuf.at[bb],
                        gsems[bb],
                    )

        return carry

    lax.fori_loop(0, my_n, body, 0)

    def drain(k, carry):
        for bb in range(FBUF):
            @pl.when(lax.rem(k, FBUF) == bb)
            def _():
                pltpu.make_async_copy(
                    tbuf.at[bb], out_hbm.at[pl.ds(col0(k), PADDED)], ssems[bb]
                ).wait()

        return carry

    lax.fori_loop(lax.max(my_n - FBUF, 0), my_n, drain, 0)

    @pl.when(wid == NW - 1)
    def _():
        pltpu.sync_copy(tt_hbm.at[:, pl.ds(TFULL * PADDED, TREM)], rbuf)

        def tblk(j0, carry2):
            rv = j0 * LANES + iota
            for d0 in range(EMB_DIM // LANES):
                for r in range(LANES):
                    cv = d0 * LANES + ((iota + r) & (LANES - 1))
                    v = plsc.load_gather(rbuf, [cv, rv])
                    plsc.store_scatter(rtbuf, [rv, cv], v)
            return carry2

        lax.fori_loop(0, TREM // LANES, tblk, 0)
        pltpu.sync_copy(rtbuf, out_hbm.at[pl.ds(TFULL * PADDED, TREM)])


@functools.partial(
    pl.kernel,
    out_type=jax.ShapeDtypeStruct((SEQ, EMB_DIM, BATCH), jnp.float32),
    mesh=_mesh,
    compiler_params=pltpu.CompilerParams(
        use_tc_tiling_on_sc=True, needs_layout_passes=False
    ),
    scratch_types=[
        pltpu.VMEM((SEQ * BBLK,), jnp.int32),
        pltpu.VMEM((NBUF, BBLK), jnp.int32),
        pltpu.VMEM((NBUF, BBLK, PADDED), jnp.float32),
        pltpu.VMEM((NBUF, EMB_DIM, BBLK), jnp.float32),
        pltpu.SemaphoreType.DMA,
        [pltpu.SemaphoreType.DMA] * NBUF,
        [pltpu.SemaphoreType.DMA] * NBUF,
    ],
)
def _gather_kernel(
    ids_hbm, table_hbm, out_hbm, idx_v, ivc, gbuf, tbuf, isem, gsems, ssems
):
    wid = lax.axis_index("s") * NC + lax.axis_index("c")
    b0 = wid * BBLK
    base = b0 * SEQ

    pltpu.async_copy(ids_hbm.at[pl.ds(base, SEQ * BBLK)], idx_v, isem)
    pltpu.make_async_copy(ids_hbm.at[pl.ds(base, SEQ * BBLK)], idx_v, isem).wait()

    iota = lax.iota(jnp.int32, LANES)

    def make_chunk_idx(s, b):
        for j0 in range(BBLK // LANES):
            v = plsc.load_gather(idx_v, [(j0 * LANES + iota) * SEQ + s])
            ivc[b, pl.ds(j0 * LANES, LANES)] = v

    def start_gather(s, b):
        make_chunk_idx(s, b)
        pltpu.async_copy(table_hbm.at[ivc.at[b]], gbuf.at[b], gsems[b])

    def transpose_scale(b):
        def tblock(j0, carry2):
            rv = j0 * LANES + iota
            for d0 in range(EMB_DIM // LANES):
                for r in range(LANES):
                    cv = d0 * LANES + ((iota + r) & (LANES - 1))
                    v = plsc.load_gather(gbuf.at[b], [rv, cv])
                    plsc.store_scatter(tbuf.at[b], [cv, rv], v * SCALE)
            return carry2

        lax.fori_loop(0, BBLK // LANES, tblock, 0, unroll=4)

    for b in range(NBUF):
        start_gather(b, b)

    def outer(g0, carry):
        for b in range(NBUF):
            s = g0 * NBUF + b
            pltpu.make_async_copy(
                table_hbm.at[ivc.at[b]], gbuf.at[b], gsems[b]
            ).wait()

            @pl.when(g0 > 0)
            def _():
                pltpu.make_async_copy(
                    tbuf.at[b], out_hbm.at[s - NBUF, :, pl.ds(b0, BBLK)], ssems[b]
                ).wait()

            transpose_scale(b)

            pltpu.async_copy(
                tbuf.at[b], out_hbm.at[s, :, pl.ds(b0, BBLK)], ssems[b]
            )

            @pl.when(s + NBUF < SEQ)
            def _():
                start_gather(s + NBUF, b)

        return carry

    lax.fori_loop(0, SEQ // NBUF, outer, 0)

    for s in range((SEQ // NBUF) * NBUF, SEQ):
        b = s % NBUF
        pltpu.make_async_copy(table_hbm.at[ivc.at[b]], gbuf.at[b], gsems[b]).wait()
        pltpu.make_async_copy(
            tbuf.at[b], out_hbm.at[s - NBUF, :, pl.ds(b0, BBLK)], ssems[b]
        ).wait()

        transpose_scale(b)
        pltpu.async_copy(tbuf.at[b], out_hbm.at[s, :, pl.ds(b0, BBLK)], ssems[b])

    for s in range(SEQ - NBUF, SEQ):
        b = s % NBUF
        pltpu.make_async_copy(
            tbuf.at[b], out_hbm.at[s, :, pl.ds(b0, BBLK)], ssems[b]
        ).wait()


def kernel(ids, table):
    table_p = _format_table(table.T)
    flat_ids = ids.astype(jnp.int32).reshape(BATCH * SEQ)
    out_t = _gather_kernel(flat_ids, table_p)
    return out_t.transpose(2, 0, 1)


# speedup vs baseline: 1.5738x; 1.5738x over previous
import functools
import math

import jax
import jax.numpy as jnp
from jax import lax
from jax.experimental import pallas as pl
from jax.experimental.pallas import tpu as pltpu
from jax.experimental.pallas import tpu_sc as plsc

VOCAB = 1000000
EMB_DIM = 64
PADDED = 128
BATCH = 4096
SEQ = 200

NC = 2
NS = 16
NW = NC * NS
LANES = 16

BBLK = BATCH // NW
NBUF = 3
SCALE = math.sqrt(EMB_DIM)

_mesh = plsc.VectorSubcoreMesh(
    core_axis_name="c", subcore_axis_name="s", num_cores=NC, num_subcores=NS
)


TCHK = 7813
TFULL = VOCAB // PADDED
TREM = VOCAB - TFULL * PADDED
FBUF = 3


@functools.partial(
    pl.kernel,
    out_type=jax.ShapeDtypeStruct((VOCAB, PADDED), jnp.float32),
    mesh=_mesh,
    compiler_params=pltpu.CompilerParams(
        use_tc_tiling_on_sc=True, needs_layout_passes=False
    ),
    scratch_types=[
        pltpu.VMEM((FBUF, EMB_DIM, PADDED), jnp.float32),
        pltpu.VMEM((FBUF, PADDED, PADDED), jnp.float32),
        pltpu.VMEM((EMB_DIM, TREM), jnp.float32),
        pltpu.VMEM((TREM, PADDED), jnp.float32),
        [pltpu.SemaphoreType.DMA] * FBUF,
        [pltpu.SemaphoreType.DMA] * FBUF,
    ],
)
def _format_table(tt_hbm, out_hbm, gbuf, tbuf, rbuf, rtbuf, gsems, ssems):
    wid = lax.axis_index("s") * NC + lax.axis_index("c")
    my_n = TFULL // NW + jnp.where(wid < TFULL % NW, 1, 0)

    iota = lax.iota(jnp.int32, LANES)

    def col0(k):
        return (wid + k * NW) * PADDED

    def transpose_block(bb, ngroups):
        def tblk(j0, carry2):
            rv = j0 * LANES + iota
            for d0 in range(EMB_DIM // LANES):
                for r in range(LANES):
                    cv = d0 * LANES + ((iota + r) & (LANES - 1))
                    v = plsc.load_gather(gbuf.at[bb], [cv, rv])
                    plsc.store_scatter(tbuf.at[bb], [rv, cv], v)
            return carry2

        lax.fori_loop(0, ngroups, tblk, 0)

    for b in range(FBUF):
        @pl.when(b < my_n)
        def _():
            pltpu.async_copy(
                tt_hbm.at[:, pl.ds(col0(b), PADDED)], gbuf.at[b], gsems[b]
            )

    def body(k, carry):
        for bb in range(FBUF):
            @pl.when(lax.rem(k, FBUF) == bb)
            def _():
                pltpu.make_async_copy(
                    tt_hbm.at[:, pl.ds(col0(k), PADDED)], gbuf.at[bb], gsems[bb]
                ).wait()

                @pl.when(k >= FBUF)
                def _():
                    pltpu.make_async_copy(
                        tbuf.at[bb],
                        out_hbm.at[pl.ds(col0(k - FBUF), PADDED)],
                        ssems[bb],
                    ).wait()

                transpose_block(bb, PADDED // LANES)

                pltpu.async_copy(
                    tbuf.at[bb], out_hbm.at[pl.ds(col0(k), PADDED)], ssems[bb]
                )

                @pl.when(k + FBUF < my_n)
                def _():
                    pltpu.async_copy(
                        tt_hbm.at[:, pl.ds(col0(k + FBUF), PADDED)],
                        gbuf.at[bb],
                        gsems[bb],
                    )

        return carry

    lax.fori_loop(0, my_n, body, 0)

    def drain(k, carry):
        for bb in range(FBUF):
            @pl.when(lax.rem(k, FBUF) == bb)
            def _():
                pltpu.make_async_copy(
                    tbuf.at[bb], out_hbm.at[pl.ds(col0(k), PADDED)], ssems[bb]
                ).wait()

        return carry

    lax.fori_loop(lax.max(my_n - FBUF, 0), my_n, drain, 0)

    @pl.when(wid == NW - 1)
    def _():
        pltpu.sync_copy(tt_hbm.at[:, pl.ds(TFULL * PADDED, TREM)], rbuf)

        def tblk(j0, carry2):
            rv = j0 * LANES + iota
            for d0 in range(EMB_DIM // LANES):
                for r in range(LANES):
                    cv = d0 * LANES + ((iota + r) & (LANES - 1))
                    v = plsc.load_gather(rbuf, [cv, rv])
                    plsc.store_scatter(rtbuf, [rv, cv], v)
            return carry2

        lax.fori_loop(0, TREM // LANES, tblk, 0)
        pltpu.sync_copy(rtbuf, out_hbm.at[pl.ds(TFULL * PADDED, TREM)])


@functools.partial(
    pl.kernel,
    out_type=jax.ShapeDtypeStruct((SEQ, EMB_DIM, BATCH), jnp.float32),
    mesh=_mesh,
    compiler_params=pltpu.CompilerParams(
        use_tc_tiling_on_sc=True, needs_layout_passes=False
    ),
    scratch_types=[
        pltpu.VMEM((SEQ * BBLK,), jnp.int32),
        pltpu.VMEM((NBUF, BBLK), jnp.int32),
        pltpu.VMEM((NBUF, BBLK, PADDED), jnp.float32),
        pltpu.VMEM((NBUF, EMB_DIM, BBLK), jnp.float32),
        pltpu.SemaphoreType.DMA,
        [pltpu.SemaphoreType.DMA] * NBUF,
        [pltpu.SemaphoreType.DMA] * NBUF,
    ],
)
def _gather_kernel(
    ids_hbm, table_hbm, out_hbm, idx_v, ivc, gbuf, tbuf, isem, gsems, ssems
):
    wid = lax.axis_index("s") * NC + lax.axis_index("c")
    b0 = wid * BBLK
    base = b0 * SEQ

    pltpu.async_copy(ids_hbm.at[pl.ds(base, SEQ * BBLK)], idx_v, isem)
    pltpu.make_async_copy(ids_hbm.at[pl.ds(base, SEQ * BBLK)], idx_v, isem).wait()

    iota = lax.iota(jnp.int32, LANES)

    def make_chunk_idx(s, b):
        for j0 in range(BBLK // LANES):
            v = plsc.load_gather(idx_v, [(j0 * LANES + iota) * SEQ + s])
            ivc[b, pl.ds(j0 * LANES, LANES)] = v

    def start_gather(s, b):
        make_chunk_idx(s, b)
        pltpu.async_copy(table_hbm.at[ivc.at[b]], gbuf.at[b], gsems[b])

    def transpose_scale(b):
        def tblock(j0, carry2):
            rv = j0 * LANES + iota
            for d0 in range(EMB_DIM // LANES):
                for r in range(LANES):
                    cv = d0 * LANES + ((iota + r) & (LANES - 1))
                    v = plsc.load_gather(gbuf.at[b], [rv, cv])
                    plsc.store_scatter(tbuf.at[b], [cv, rv], v * SCALE)
            return carry2

        lax.fori_loop(0, BBLK // LANES, tblock, 0, unroll=2)

    for b in range(NBUF):
        start_gather(b, b)

    def outer(g0, carry):
        for b in range(NBUF):
            s = g0 * NBUF + b
            pltpu.make_async_copy(
                table_hbm.at[ivc.at[b]], gbuf.at[b], gsems[b]
            ).wait()

            @pl.when(g0 > 0)
            def _():
                pltpu.make_async_copy(
                    tbuf.at[b], out_hbm.at[s - NBUF, :, pl.ds(b0, BBLK)], ssems[b]
                ).wait()

            transpose_scale(b)

            pltpu.async_copy(
                tbuf.at[b], out_hbm.at[s, :, pl.ds(b0, BBLK)], ssems[b]
            )

            @pl.when(s + NBUF < SEQ)
            def _():
                start_gather(s + NBUF, b)

        return carry

    lax.fori_loop(0, SEQ // NBUF, outer, 0)

    for s in range((SEQ // NBUF) * NBUF, SEQ):
        b = s % NBUF
        pltpu.make_async_copy(table_hbm.at[ivc.at[b]], gbuf.at[b], gsems[b]).wait()
        pltpu.make_async_copy(
            tbuf.at[b], out_hbm.at[s - NBUF, :, pl.ds(b0, BBLK)], ssems[b]
        ).wait()

        transpose_scale(b)
        pltpu.async_copy(tbuf.at[b], out_hbm.at[s, :, pl.ds(b0, BBLK)], ssems[b])

    for s in range(SEQ - NBUF, SEQ):
        b = s % NBUF
        pltpu.make_async_copy(
            tbuf.at[b], out_hbm.at[s, :, pl.ds(b0, BBLK)], ssems[b]
        ).wait()


def kernel(ids, table):
    table_p = _format_table(table.T)
    flat_ids = ids.astype(jnp.int32).reshape(BATCH * SEQ)
    out_t = _gather_kernel(flat_ids, table_p)
    return out_t.transpose(2, 0, 1)


# speedup vs baseline: 1.7978x; 1.1423x over previous
import functools
import math

import jax
import jax.numpy as jnp
from jax import lax
from jax.experimental import pallas as pl
from jax.experimental.pallas import tpu as pltpu
from jax.experimental.pallas import tpu_sc as plsc

VOCAB = 1000000
EMB_DIM = 64
PADDED = 128
BATCH = 4096
SEQ = 200

NC = 2
NS = 16
NW = NC * NS
LANES = 16

BBLK = BATCH // NW
NBUF = 3
SCALE = math.sqrt(EMB_DIM)

_mesh = plsc.VectorSubcoreMesh(
    core_axis_name="c", subcore_axis_name="s", num_cores=NC, num_subcores=NS
)


@functools.partial(
    pl.kernel,
    out_type=jax.ShapeDtypeStruct((SEQ, EMB_DIM, BATCH), jnp.float32),
    mesh=_mesh,
    compiler_params=pltpu.CompilerParams(
        use_tc_tiling_on_sc=True, needs_layout_passes=False
    ),
    scratch_types=[
        pltpu.VMEM((SEQ * BBLK,), jnp.int32),
        pltpu.VMEM((NBUF, BBLK), jnp.int32),
        pltpu.VMEM((NBUF, BBLK, PADDED), jnp.float32),
        pltpu.VMEM((NBUF, EMB_DIM, BBLK), jnp.float32),
        pltpu.SemaphoreType.DMA,
        [pltpu.SemaphoreType.DMA] * NBUF,
        [pltpu.SemaphoreType.DMA] * NBUF,
    ],
)
def _gather_kernel(
    ids_hbm, table_hbm, out_hbm, idx_v, ivc, gbuf, tbuf, isem, gsems, ssems
):
    wid = lax.axis_index("s") * NC + lax.axis_index("c")
    b0 = wid * BBLK
    base = b0 * SEQ

    pltpu.async_copy(ids_hbm.at[pl.ds(base, SEQ * BBLK)], idx_v, isem)
    pltpu.make_async_copy(ids_hbm.at[pl.ds(base, SEQ * BBLK)], idx_v, isem).wait()

    iota = lax.iota(jnp.int32, LANES)

    def make_chunk_idx(s, b):
        for j0 in range(BBLK // LANES):
            v = plsc.load_gather(idx_v, [(j0 * LANES + iota) * SEQ + s])
            ivc[b, pl.ds(j0 * LANES, LANES)] = v

    def start_gather(s, b):
        make_chunk_idx(s, b)
        pltpu.async_copy(table_hbm.at[ivc.at[b]], gbuf.at[b], gsems[b])

    def transpose_scale(b):
        def tblock(j0, carry2):
            rv = j0 * LANES + iota
            for d0 in range(EMB_DIM // LANES):
                for r in range(LANES):
                    cv = d0 * LANES + ((iota + r) & (LANES - 1))
                    v = plsc.load_gather(gbuf.at[b], [rv, cv])
                    plsc.store_scatter(tbuf.at[b], [cv, rv], v * SCALE)
            return carry2

        lax.fori_loop(0, BBLK // LANES, tblock, 0, unroll=2)

    for b in range(NBUF):
        start_gather(b, b)

    def outer(g0, carry):
        for b in range(NBUF):
            s = g0 * NBUF + b
            pltpu.make_async_copy(
                table_hbm.at[ivc.at[b]], gbuf.at[b], gsems[b]
            ).wait()

            @pl.when(g0 > 0)
            def _():
                pltpu.make_async_copy(
                    tbuf.at[b], out_hbm.at[s - NBUF, :, pl.ds(b0, BBLK)], ssems[b]
                ).wait()

            transpose_scale(b)

            pltpu.async_copy(
                tbuf.at[b], out_hbm.at[s, :, pl.ds(b0, BBLK)], ssems[b]
            )

            @pl.when(s + NBUF < SEQ)
            def _():
                start_gather(s + NBUF, b)

        return carry

    lax.fori_loop(0, SEQ // NBUF, outer, 0)

    for s in range((SEQ // NBUF) * NBUF, SEQ):
        b = s % NBUF
        pltpu.make_async_copy(table_hbm.at[ivc.at[b]], gbuf.at[b], gsems[b]).wait()
        pltpu.make_async_copy(
            tbuf.at[b], out_hbm.at[s - NBUF, :, pl.ds(b0, BBLK)], ssems[b]
        ).wait()

        transpose_scale(b)
        pltpu.async_copy(tbuf.at[b], out_hbm.at[s, :, pl.ds(b0, BBLK)], ssems[b])

    for s in range(SEQ - NBUF, SEQ):
        b = s % NBUF
        pltpu.make_async_copy(
            tbuf.at[b], out_hbm.at[s, :, pl.ds(b0, BBLK)], ssems[b]
        ).wait()


def kernel(ids, table):
    table_p = jnp.pad(table, ((0, 0), (0, PADDED - EMB_DIM)))
    flat_ids = ids.astype(jnp.int32).reshape(BATCH * SEQ)
    out_t = _gather_kernel(flat_ids, table_p)
    return out_t.transpose(2, 0, 1)
